# baseline (speedup 1.0000x reference)
import jax
import jax.numpy as jnp
from jax import lax
from jax.experimental import pallas as pl
from jax.experimental.pallas import tpu as pltpu
from jax.experimental.pallas import tpu_sc as plsc

B = 4096
S = 200
F = 32
E_VAR = 32
E_TIME = 16
E_SECT = 32
E_OUT = F + E_VAR + E_TIME + E_SECT
NC, NS = 2, 16
BB = B // NS
NTASK = S // NC
TSTR = F + 1


def _t2v_body(t_ref, wt_ref, bt_ref, out_ref):
    xa = t_ref[...] * wt_ref[...] + bt_ref[...]
    row = lax.broadcasted_iota(jnp.int32, (E_TIME, 2), 0)
    val = jnp.where(row == 0, xa, jnp.sin(xa))
    lane = lax.broadcasted_iota(jnp.int32, (E_TIME, 2 * BB), 1)
    out_ref[...] = jnp.where(
        lane < BB,
        jnp.broadcast_to(val[:, 0:1], (E_TIME, 2 * BB)),
        jnp.broadcast_to(val[:, 1:2], (E_TIME, 2 * BB)),
    )


def _time_pattern(t2, t2v_w, t2v_b):
    return pl.pallas_call(
        _t2v_body,
        out_shape=jax.ShapeDtypeStruct((E_TIME, 2 * BB), jnp.float32),
    )(t2.reshape(1, 2), t2v_w.T, t2v_b.T)


def _sc_body(seq_hbm, vidx_hbm, sidx_hbm, var_hbm, sect_hbm, pat_hbm, out_hbm,
             vidx_v, sidx_v, var_v, sect_v, out_v,
             sem_i, sem_q, sem_w):
    par = lax.axis_index("c")
    bb0 = pl.multiple_of(lax.axis_index("s") * BB, BB)
    pltpu.sync_copy(var_hbm, var_v)
    pltpu.sync_copy(sect_hbm, sect_v)
    poff = pl.multiple_of(par * BB, BB)
    for b in range(3):
        pltpu.sync_copy(pat_hbm.at[:, pl.ds(poff, BB)],
                        out_v.at[b, pl.ds(F + E_VAR, E_TIME), :])

    def row_of(j):
        return par + 2 * j

    def start_loads(j, b):
        pltpu.async_copy(vidx_hbm.at[row_of(j), pl.ds(bb0, BB)],
                         vidx_v.at[b], sem_i)
        pltpu.async_copy(sidx_hbm.at[row_of(j), pl.ds(bb0, BB)],
                         sidx_v.at[b], sem_i)
        pltpu.async_copy(seq_hbm.at[row_of(j), :, pl.ds(bb0, BB)],
                         out_v.at[b, pl.ds(0, F), :], sem_q)

    def wait_loads(j, b):
        pltpu.make_async_copy(vidx_hbm.at[row_of(j), pl.ds(bb0, BB)],
                              vidx_v.at[b], sem_i).wait()
        pltpu.make_async_copy(sidx_hbm.at[row_of(j), pl.ds(bb0, BB)],
                              sidx_v.at[b], sem_i).wait()
        pltpu.make_async_copy(seq_hbm.at[row_of(j), :, pl.ds(bb0, BB)],
                              out_v.at[b, pl.ds(0, F), :], sem_q).wait()

    def wait_write(j, b):
        pltpu.make_async_copy(out_v.at[b],
                              out_hbm.at[row_of(j), :, pl.ds(bb0, BB)],
                              sem_w).wait()

    start_loads(0, 0)

    def task(j, carry):
        pb = lax.rem(j, 3)
        nb = lax.rem(j + 1, 3)

        @pl.when(j + 1 < NTASK)
        def _():
            @pl.when(j >= 2)
            def _():
                wait_write(j - 2, nb)
            start_loads(j + 1, nb)

        wait_loads(j, pb)

        @plsc.parallel_loop(0, BB, step=16, unroll=2)
        def _group(lo):
            bi = vidx_v[pb, pl.ds(lo, 16)] * TSTR
            si = sidx_v[pb, pl.ds(lo, 16)] * TSTR
            for e in range(F):
                out_v[pb, F + e, pl.ds(lo, 16)] = (
                    plsc.load_gather(var_v, [bi + e]))
                out_v[pb, F + E_VAR + E_TIME + e, pl.ds(lo, 16)] = (
                    plsc.load_gather(sect_v, [si + e]))

        pltpu.async_copy(out_v.at[pb],
                         out_hbm.at[row_of(j), :, pl.ds(bb0, BB)], sem_w)
        return carry

    lax.fori_loop(0, NTASK, task, 0)
    wait_write(NTASK - 2, (NTASK - 2) % 3)
    wait_write(NTASK - 1, (NTASK - 1) % 3)


_sc_call = pl.kernel(
    _sc_body,
    out_type=jax.ShapeDtypeStruct((S, E_OUT, B), jnp.float32),
    mesh=plsc.VectorSubcoreMesh(
        core_axis_name="c", subcore_axis_name="s",
        num_cores=NC, num_subcores=NS),
    scratch_types=[
        pltpu.VMEM((3, BB), jnp.int32),
        pltpu.VMEM((3, BB), jnp.int32),
        pltpu.VMEM((100 * TSTR,), jnp.float32),
        pltpu.VMEM((100 * TSTR,), jnp.float32),
        pltpu.VMEM((3, E_OUT, BB), jnp.float32),
        pltpu.SemaphoreType.DMA,
        pltpu.SemaphoreType.DMA,
        pltpu.SemaphoreType.DMA,
    ],
    compiler_params=pltpu.CompilerParams(use_tc_tiling_on_sc=True,
                                         needs_layout_passes=False),
)


def kernel(sequence, time_index_sequence, variable_index_sequence,
           sector_index_sequence, var_table, sect_table, t2v_w, t2v_b):
    t2 = time_index_sequence[0, :2].astype(jnp.float32)
    pattern = _time_pattern(t2, t2v_w, t2v_b)
    seqT = jnp.transpose(sequence, (1, 2, 0))
    vidxT = variable_index_sequence.T
    sidxT = sector_index_sequence.T
    varp = jnp.pad(var_table, ((0, 0), (0, TSTR - F))).reshape(100 * TSTR)
    sectp = jnp.pad(sect_table, ((0, 0), (0, TSTR - F))).reshape(100 * TSTR)
    outT = _sc_call(seqT, vidxT, sidxT, varp, sectp, pattern)
    return jnp.transpose(outT, (2, 0, 1))


# speedup vs baseline: 1.0020x; 1.0020x over previous
import jax
import jax.numpy as jnp
from jax import lax
from jax.experimental import pallas as pl
from jax.experimental.pallas import tpu as pltpu
from jax.experimental.pallas import tpu_sc as plsc

B = 4096
S = 200
F = 32
E_VAR = 32
E_TIME = 16
E_SECT = 32
E_OUT = F + E_VAR + E_TIME + E_SECT
NC, NS = 2, 16
BB = B // NS
NTASK = S // NC
TSTR = F + 1


def _t2v_body(t_ref, wt_ref, bt_ref, out_ref):
    xa = t_ref[...] * wt_ref[...] + bt_ref[...]
    row = lax.broadcasted_iota(jnp.int32, (E_TIME, 2), 0)
    val = jnp.where(row == 0, xa, jnp.sin(xa))
    lane = lax.broadcasted_iota(jnp.int32, (E_TIME, 2 * BB), 1)
    out_ref[...] = jnp.where(
        lane < BB,
        jnp.broadcast_to(val[:, 0:1], (E_TIME, 2 * BB)),
        jnp.broadcast_to(val[:, 1:2], (E_TIME, 2 * BB)),
    )


def _time_pattern(t2, t2v_w, t2v_b):
    return pl.pallas_call(
        _t2v_body,
        out_shape=jax.ShapeDtypeStruct((E_TIME, 2 * BB), jnp.float32),
    )(t2.reshape(1, 2), t2v_w.T, t2v_b.T)


def _sc_body(seq_hbm, vidx_hbm, sidx_hbm, var_hbm, sect_hbm, pat_hbm, out_hbm,
             vidx_v, sidx_v, var_v, sect_v, out_v,
             sem_i, sem_q, sem_w):
    par = lax.axis_index("c")
    bb0 = pl.multiple_of(lax.axis_index("s") * BB, BB)
    pltpu.sync_copy(var_hbm, var_v)
    pltpu.sync_copy(sect_hbm, sect_v)
    poff = pl.multiple_of(par * BB, BB)
    for b in range(2):
        pltpu.sync_copy(pat_hbm.at[:, pl.ds(poff, BB)],
                        out_v.at[b, pl.ds(F + E_VAR, E_TIME), :])

    def row_of(j):
        return par + 2 * j

    def start_loads(j, b):
        pltpu.async_copy(vidx_hbm.at[row_of(j), pl.ds(bb0, BB)],
                         vidx_v.at[b], sem_i)
        pltpu.async_copy(sidx_hbm.at[row_of(j), pl.ds(bb0, BB)],
                         sidx_v.at[b], sem_i)
        pltpu.async_copy(seq_hbm.at[row_of(j), :, pl.ds(bb0, BB)],
                         out_v.at[b, pl.ds(0, F), :], sem_q)

    def wait_loads(j, b):
        pltpu.make_async_copy(vidx_hbm.at[row_of(j), pl.ds(bb0, BB)],
                              vidx_v.at[b], sem_i).wait()
        pltpu.make_async_copy(sidx_hbm.at[row_of(j), pl.ds(bb0, BB)],
                              sidx_v.at[b], sem_i).wait()
        pltpu.make_async_copy(seq_hbm.at[row_of(j), :, pl.ds(bb0, BB)],
                              out_v.at[b, pl.ds(0, F), :], sem_q).wait()

    def wait_write(j, b):
        pltpu.make_async_copy(out_v.at[b],
                              out_hbm.at[row_of(j), :, pl.ds(bb0, BB)],
                              sem_w).wait()

    start_loads(0, 0)

    def task(j, carry):
        pb = lax.rem(j, 2)

        @pl.when(j + 1 < NTASK)
        def _():
            @pl.when(j >= 1)
            def _():
                wait_write(j - 1, 1 - pb)
            start_loads(j + 1, 1 - pb)

        wait_loads(j, pb)

        @plsc.parallel_loop(0, BB, step=16, unroll=2)
        def _group(lo):
            bi = vidx_v[pb, pl.ds(lo, 16)] * TSTR
            si = sidx_v[pb, pl.ds(lo, 16)] * TSTR
            for e in range(F):
                out_v[pb, F + e, pl.ds(lo, 16)] = (
                    plsc.load_gather(var_v, [bi + e]))
                out_v[pb, F + E_VAR + E_TIME + e, pl.ds(lo, 16)] = (
                    plsc.load_gather(sect_v, [si + e]))

        pltpu.async_copy(out_v.at[pb],
                         out_hbm.at[row_of(j), :, pl.ds(bb0, BB)], sem_w)
        return carry

    lax.fori_loop(0, NTASK, task, 0)
    wait_write(NTASK - 1, (NTASK - 1) % 2)


_sc_call = pl.kernel(
    _sc_body,
    out_type=jax.ShapeDtypeStruct((S, E_OUT, B), jnp.float32),
    mesh=plsc.VectorSubcoreMesh(
        core_axis_name="c", subcore_axis_name="s",
        num_cores=NC, num_subcores=NS),
    scratch_types=[
        pltpu.VMEM((2, BB), jnp.int32),
        pltpu.VMEM((2, BB), jnp.int32),
        pltpu.VMEM((100 * TSTR,), jnp.float32),
        pltpu.VMEM((100 * TSTR,), jnp.float32),
        pltpu.VMEM((2, E_OUT, BB), jnp.float32),
        pltpu.SemaphoreType.DMA,
        pltpu.SemaphoreType.DMA,
        pltpu.SemaphoreType.DMA,
    ],
    compiler_params=pltpu.CompilerParams(use_tc_tiling_on_sc=True,
                                         needs_layout_passes=False),
)


def kernel(sequence, time_index_sequence, variable_index_sequence,
           sector_index_sequence, var_table, sect_table, t2v_w, t2v_b):
    t2 = time_index_sequence[0, :2].astype(jnp.float32)
    pattern = _time_pattern(t2, t2v_w, t2v_b)
    seqT = jnp.transpose(sequence, (1, 2, 0))
    vidxT = variable_index_sequence.T
    sidxT = sector_index_sequence.T
    varp = jnp.pad(var_table, ((0, 0), (0, TSTR - F))).reshape(100 * TSTR)
    sectp = jnp.pad(sect_table, ((0, 0), (0, TSTR - F))).reshape(100 * TSTR)
    outT = _sc_call(seqT, vidxT, sidxT, varp, sectp, pattern)
    return jnp.transpose(outT, (2, 0, 1))
